# baseline (speedup 1.0000x reference)
import jax
import jax.numpy as jnp
from jax.experimental import pallas as pl
from jax.experimental.pallas import tpu as pltpu

DIM = 768
D_STATE = 16
D_CONV = 4
E = DIM * 2
EPS = 1e-6
T = 512


def _mamba_kernel(x_ref, gamma_ref, winT_ref, bin_ref, convT_ref,
                  at_ref, bt_ref, ct_ref, woutT_ref, bout_ref,
                  out_ref, xps_ref, xc_ref, y_ref, h_ref, gsa_ref, gsb_ref):
    t_idx = pl.program_id(1)

    xb = x_ref[0]
    ss = jnp.sum(xb * xb, axis=1, keepdims=True)
    rms = jnp.sqrt(ss * (1.0 / DIM))
    xn = xb * (gamma_ref[...] / (rms + EPS))
    xp = jnp.dot(xn.astype(jnp.bfloat16), winT_ref[...],
                 preferred_element_type=jnp.float32) + bin_ref[...]

    @pl.when(t_idx == 0)
    def _():
        xps_ref[0:8, :] = jnp.zeros((8, E), jnp.float32)
        h_ref[...] = jnp.zeros((D_STATE, E), jnp.bfloat16)

    @pl.when(t_idx > 0)
    def _():
        xps_ref[0:8, :] = xps_ref[T:T + 8, :]

    xps_ref[8:T + 8, :] = xp
    xc = (xps_ref[5:5 + T, :] * convT_ref[0:1, :]
          + xps_ref[6:6 + T, :] * convT_ref[1:2, :]
          + xps_ref[7:7 + T, :] * convT_ref[2:3, :]
          + xp * convT_ref[3:4, :])
    xc_ref[...] = xc

    sA = jax.nn.sigmoid(at_ref[...]).astype(jnp.bfloat16)
    sCB = jax.nn.sigmoid(bt_ref[...]) * jax.nn.sigmoid(ct_ref[...])

    rows = jax.lax.broadcasted_iota(jnp.int32, (16, 256), 0)
    cols = jax.lax.broadcasted_iota(jnp.int32, (16, 256), 1)
    sel = jnp.where(rows == cols // D_STATE, 1.0, 0.0).astype(jnp.bfloat16)

    NG = T // 16

    def steps16(grp, p, buf):
        base = pl.multiple_of(grp * 16, 16)
        xg = xc_ref[pl.ds(base, 16), :]
        for i in range(16):
            u = (sCB * xg[i:i + 1, :]).astype(jnp.bfloat16)
            p = sA * p + u
            buf[i * 16:(i + 1) * 16, :] = p
        return p

    def reduce16(grp, buf):
        base = pl.multiple_of(grp * 16, 16)
        y_ref[pl.ds(base, 16), :] = jnp.dot(
            sel, buf[...], preferred_element_type=jnp.float32)


    p = steps16(0, h_ref[...], gsa_ref)
    p = steps16(1, p, gsb_ref)
    reduce16(0, gsa_ref)

    def body(j, p):
        p = steps16(2 * j, p, gsa_ref)
        reduce16(2 * j - 1, gsb_ref)
        p = steps16(2 * j + 1, p, gsb_ref)
        reduce16(2 * j, gsa_ref)
        return p

    p_fin = jax.lax.fori_loop(1, NG // 2, body, p)
    reduce16(NG - 1, gsb_ref)
    h_ref[...] = p_fin

    y = y_ref[...]
    out = jnp.dot(y.astype(jnp.bfloat16), woutT_ref[...],
                  preferred_element_type=jnp.float32) + bout_ref[...]
    out_ref[0] = out + xb


def kernel(x, gamma, W_in, b_in, conv_w, A, Bp, C, W_out, b_out):
    B, L, _ = x.shape
    grid = (B, L // T)
    rep = lambda *_: (0, 0)
    out = pl.pallas_call(
        _mamba_kernel,
        grid=grid,
        in_specs=[
            pl.BlockSpec((1, T, DIM), lambda b, t: (b, t, 0)),
            pl.BlockSpec((1, DIM), rep),
            pl.BlockSpec((DIM, E), rep),
            pl.BlockSpec((1, E), rep),
            pl.BlockSpec((D_CONV, E), rep),
            pl.BlockSpec((D_STATE, E), rep),
            pl.BlockSpec((D_STATE, E), rep),
            pl.BlockSpec((D_STATE, E), rep),
            pl.BlockSpec((E, DIM), rep),
            pl.BlockSpec((1, DIM), rep),
        ],
        out_specs=pl.BlockSpec((1, T, DIM), lambda b, t: (b, t, 0)),
        out_shape=jax.ShapeDtypeStruct((B, L, DIM), jnp.float32),
        scratch_shapes=[
            pltpu.VMEM((T + 8, E), jnp.float32),
            pltpu.VMEM((T, E), jnp.float32),
            pltpu.VMEM((T, E), jnp.float32),
            pltpu.VMEM((D_STATE, E), jnp.bfloat16),
            pltpu.VMEM((256, E), jnp.bfloat16),
            pltpu.VMEM((256, E), jnp.bfloat16),
        ],
        compiler_params=pltpu.CompilerParams(
            dimension_semantics=("parallel", "arbitrary"),
            flags={"XLA_TPU_STORE_TO_LOAD_FORWARDING_WINDOW": 8192},
        ),
        name="mamba_block",
    )(
        x,
        gamma.reshape(1, DIM),
        W_in.T.astype(jnp.bfloat16),
        b_in.reshape(1, E),
        conv_w.T,
        A.T,
        Bp.T,
        C.T,
        W_out.T.astype(jnp.bfloat16),
        b_out.reshape(1, DIM),
    )
    return out


# speedup vs baseline: 1.0080x; 1.0080x over previous
import jax
import jax.numpy as jnp
from jax.experimental import pallas as pl
from jax.experimental.pallas import tpu as pltpu

DIM = 768
D_STATE = 16
D_CONV = 4
E = DIM * 2
EPS = 1e-6
T = 512


def _mamba_kernel(x_ref, gamma_ref, winT_ref, bin_ref, convT_ref,
                  at_ref, bt_ref, ct_ref, woutT_ref, bout_ref,
                  out_ref, xps_ref, xc_ref, y_ref, h_ref, gsa_ref, gsb_ref):
    t_idx = pl.program_id(1)

    xb = x_ref[0]
    ss = jnp.sum(xb * xb, axis=1, keepdims=True)
    rms = jnp.sqrt(ss * (1.0 / DIM))
    xn = xb * (gamma_ref[...] / (rms + EPS))
    xp = jnp.dot(xn.astype(jnp.bfloat16), winT_ref[...],
                 preferred_element_type=jnp.float32) + bin_ref[...]

    @pl.when(t_idx == 0)
    def _():
        xps_ref[0:8, :] = jnp.zeros((8, E), jnp.float32)
        h_ref[...] = jnp.zeros((D_STATE, E), jnp.bfloat16)

    @pl.when(t_idx > 0)
    def _():
        xps_ref[0:8, :] = xps_ref[T:T + 8, :]

    xps_ref[8:T + 8, :] = xp
    xc = (xps_ref[5:5 + T, :] * convT_ref[0:1, :]
          + xps_ref[6:6 + T, :] * convT_ref[1:2, :]
          + xps_ref[7:7 + T, :] * convT_ref[2:3, :]
          + xp * convT_ref[3:4, :])
    xc_ref[...] = xc

    sA = jax.nn.sigmoid(at_ref[...]).astype(jnp.bfloat16)
    sCB = jax.nn.sigmoid(bt_ref[...]) * jax.nn.sigmoid(ct_ref[...])

    G = 32

    rows = jax.lax.broadcasted_iota(jnp.int32, (G, G * 16), 0)
    cols = jax.lax.broadcasted_iota(jnp.int32, (G, G * 16), 1)
    sel = jnp.where(rows == cols // D_STATE, 1.0, 0.0).astype(jnp.bfloat16)

    NG = T // G

    def stepsG(grp, p, buf):
        base = pl.multiple_of(grp * G, G)
        xg = xc_ref[pl.ds(base, G), :]
        for i in range(G):
            u = (sCB * xg[i:i + 1, :]).astype(jnp.bfloat16)
            p = sA * p + u
            buf[i * 16:(i + 1) * 16, :] = p
        return p

    def reduceG(grp, buf):
        base = pl.multiple_of(grp * G, G)
        y_ref[pl.ds(base, G), :] = jnp.dot(
            sel, buf[...], preferred_element_type=jnp.float32)

    p = stepsG(0, h_ref[...], gsa_ref)
    p = stepsG(1, p, gsb_ref)
    reduceG(0, gsa_ref)

    def body(j, p):
        p = stepsG(2 * j, p, gsa_ref)
        reduceG(2 * j - 1, gsb_ref)
        p = stepsG(2 * j + 1, p, gsb_ref)
        reduceG(2 * j, gsa_ref)
        return p

    p_fin = jax.lax.fori_loop(1, NG // 2, body, p)
    reduceG(NG - 1, gsb_ref)
    h_ref[...] = p_fin

    y = y_ref[...]
    out = jnp.dot(y.astype(jnp.bfloat16), woutT_ref[...],
                  preferred_element_type=jnp.float32) + bout_ref[...]
    out_ref[0] = out + xb


def kernel(x, gamma, W_in, b_in, conv_w, A, Bp, C, W_out, b_out):
    B, L, _ = x.shape
    grid = (B, L // T)
    rep = lambda *_: (0, 0)
    out = pl.pallas_call(
        _mamba_kernel,
        grid=grid,
        in_specs=[
            pl.BlockSpec((1, T, DIM), lambda b, t: (b, t, 0)),
            pl.BlockSpec((1, DIM), rep),
            pl.BlockSpec((DIM, E), rep),
            pl.BlockSpec((1, E), rep),
            pl.BlockSpec((D_CONV, E), rep),
            pl.BlockSpec((D_STATE, E), rep),
            pl.BlockSpec((D_STATE, E), rep),
            pl.BlockSpec((D_STATE, E), rep),
            pl.BlockSpec((E, DIM), rep),
            pl.BlockSpec((1, DIM), rep),
        ],
        out_specs=pl.BlockSpec((1, T, DIM), lambda b, t: (b, t, 0)),
        out_shape=jax.ShapeDtypeStruct((B, L, DIM), jnp.float32),
        scratch_shapes=[
            pltpu.VMEM((T + 8, E), jnp.float32),
            pltpu.VMEM((T, E), jnp.float32),
            pltpu.VMEM((T, E), jnp.float32),
            pltpu.VMEM((D_STATE, E), jnp.bfloat16),
            pltpu.VMEM((512, E), jnp.bfloat16),
            pltpu.VMEM((512, E), jnp.bfloat16),
        ],
        compiler_params=pltpu.CompilerParams(
            dimension_semantics=("parallel", "arbitrary"),
            flags={"XLA_TPU_STORE_TO_LOAD_FORWARDING_WINDOW": 8192},
        ),
        name="mamba_block",
    )(
        x,
        gamma.reshape(1, DIM),
        W_in.T.astype(jnp.bfloat16),
        b_in.reshape(1, E),
        conv_w.T,
        A.T,
        Bp.T,
        C.T,
        W_out.T.astype(jnp.bfloat16),
        b_out.reshape(1, DIM),
    )
    return out


# speedup vs baseline: 1.0086x; 1.0006x over previous
import jax
import jax.numpy as jnp
from jax.experimental import pallas as pl
from jax.experimental.pallas import tpu as pltpu

DIM = 768
D_STATE = 16
D_CONV = 4
E = DIM * 2
EPS = 1e-6
T = 512


def _mamba_kernel(x_ref, gamma_ref, winT_ref, bin_ref, convT_ref,
                  at_ref, bt_ref, ct_ref, woutT_ref, bout_ref,
                  out_ref, xps_ref, xc_ref, y_ref, h_ref, gsa_ref, gsb_ref):
    t_idx = pl.program_id(1)

    xb = x_ref[0]
    ss = jnp.sum(xb * xb, axis=1, keepdims=True)
    rms = jnp.sqrt(ss * (1.0 / DIM))
    xn = xb * (gamma_ref[...] / (rms + EPS))
    xp = jnp.dot(xn.astype(jnp.bfloat16), winT_ref[...],
                 preferred_element_type=jnp.float32) + bin_ref[...]

    @pl.when(t_idx == 0)
    def _():
        xps_ref[0:8, :] = jnp.zeros((8, E), jnp.float32)
        h_ref[...] = jnp.zeros((D_STATE, E), jnp.bfloat16)

    @pl.when(t_idx > 0)
    def _():
        xps_ref[0:8, :] = xps_ref[T:T + 8, :]

    xps_ref[8:T + 8, :] = xp
    xc = (xps_ref[5:5 + T, :] * convT_ref[0:1, :]
          + xps_ref[6:6 + T, :] * convT_ref[1:2, :]
          + xps_ref[7:7 + T, :] * convT_ref[2:3, :]
          + xp * convT_ref[3:4, :])
    xc_ref[...] = xc

    sA = jax.nn.sigmoid(at_ref[...]).astype(jnp.bfloat16)
    sCB = jax.nn.sigmoid(bt_ref[...]) * jax.nn.sigmoid(ct_ref[...])

    G = 32

    rows = jax.lax.broadcasted_iota(jnp.int32, (G, G * 16), 0)
    cols = jax.lax.broadcasted_iota(jnp.int32, (G, G * 16), 1)
    sel = jnp.where(rows == cols // D_STATE, 1.0, 0.0).astype(jnp.bfloat16)

    NG = T // G

    def stepsG(grp, p, buf):
        base = pl.multiple_of(grp * G, G)
        xg = xc_ref[pl.ds(base, G), :]
        for i in range(G):
            u = (sCB * xg[i:i + 1, :]).astype(jnp.bfloat16)
            p = sA * p + u
            buf[i * 16:(i + 1) * 16, :] = p
        return p

    def reduceG(grp, buf):
        base = pl.multiple_of(grp * G, G)
        y_ref[pl.ds(base, G), :] = jnp.dot(
            sel, buf[...], preferred_element_type=jnp.float32)

    p = stepsG(0, h_ref[...], gsa_ref)
    p = stepsG(1, p, gsb_ref)
    reduceG(0, gsa_ref)

    def body(j, p):
        p = stepsG(2 * j, p, gsa_ref)
        reduceG(2 * j - 1, gsb_ref)
        p = stepsG(2 * j + 1, p, gsb_ref)
        reduceG(2 * j, gsa_ref)
        return p

    p_fin = jax.lax.fori_loop(1, NG // 2, body, p)
    reduceG(NG - 1, gsb_ref)
    h_ref[...] = p_fin

    y = y_ref[...]
    out = jnp.dot(y.astype(jnp.bfloat16), woutT_ref[...],
                  preferred_element_type=jnp.float32) + bout_ref[...]
    out_ref[0] = out + xb


def kernel(x, gamma, W_in, b_in, conv_w, A, Bp, C, W_out, b_out):
    B, L, _ = x.shape
    grid = (B, L // T)
    rep = lambda *_: (0, 0)
    out = pl.pallas_call(
        _mamba_kernel,
        grid=grid,
        in_specs=[
            pl.BlockSpec((1, T, DIM), lambda b, t: (b, t, 0)),
            pl.BlockSpec((1, DIM), rep),
            pl.BlockSpec((DIM, E), rep),
            pl.BlockSpec((1, E), rep),
            pl.BlockSpec((D_CONV, E), rep),
            pl.BlockSpec((D_STATE, E), rep),
            pl.BlockSpec((D_STATE, E), rep),
            pl.BlockSpec((D_STATE, E), rep),
            pl.BlockSpec((E, DIM), rep),
            pl.BlockSpec((1, DIM), rep),
        ],
        out_specs=pl.BlockSpec((1, T, DIM), lambda b, t: (b, t, 0)),
        out_shape=jax.ShapeDtypeStruct((B, L, DIM), jnp.float32),
        scratch_shapes=[
            pltpu.VMEM((T + 8, E), jnp.float32),
            pltpu.VMEM((T, E), jnp.float32),
            pltpu.VMEM((T, E), jnp.float32),
            pltpu.VMEM((D_STATE, E), jnp.bfloat16),
            pltpu.VMEM((512, E), jnp.bfloat16),
            pltpu.VMEM((512, E), jnp.bfloat16),
        ],
        compiler_params=pltpu.CompilerParams(
            dimension_semantics=("parallel", "arbitrary"),
        ),
        name="mamba_block",
    )(
        x,
        gamma.reshape(1, DIM),
        W_in.T.astype(jnp.bfloat16),
        b_in.reshape(1, E),
        conv_w.T,
        A.T,
        Bp.T,
        C.T,
        W_out.T.astype(jnp.bfloat16),
        b_out.reshape(1, DIM),
    )
    return out
